# Initial kernel scaffold; baseline (speedup 1.0000x reference)
import jax
import jax.numpy as jnp
from jax.experimental import pallas as pl


def kernel(node_ids, rel_ids, center_mol_idx, edge_index, node_table, rel_table, lin_W, lin_b, conv1_W, conv1_b, conv2_W, conv2_b, ec_W, ec_b, mo_W, mo_b, nc_W, nc_b):
    raise NotImplementedError("write your pallas kernel here")


# speedup vs baseline: 3.9853x; 3.9853x over previous
import functools

import jax
import jax.numpy as jnp
from jax import lax
from jax.experimental import pallas as pl
from jax.experimental.pallas import tpu as pltpu
from jax.experimental.pallas import tpu_sc as plsc

N = 10000
E = 320000
P = 10240
D = 128
HD = 64
NC = 2
NS = 16
NW = NC * NS
EW = E // NW
CH = 80
NCH = EW // CH


def _mesh():
    return plsc.VectorSubcoreMesh(core_axis_name="c", subcore_axis_name="s",
                                  num_cores=NC, num_subcores=NS)


def _sc_gather_rows(table, idx):
    n_out = idx.shape[0]
    d = table.shape[1]
    bw = n_out // NW
    ch = min(bw, CH)
    nch = bw // ch

    @functools.partial(
        pl.kernel,
        out_type=jax.ShapeDtypeStruct((n_out, d), jnp.float32),
        mesh=_mesh(),
        scratch_types=[pltpu.VMEM((ch,), jnp.int32),
                       pltpu.VMEM((ch, d), jnp.float32),
                       pltpu.SemaphoreType.DMA],
    )
    def k(tbl, ix, out, idx_v, rows_v, sem):
        base = (lax.axis_index("s") * NC + lax.axis_index("c")) * bw

        def chunk(c, carry):
            b = base + c * ch
            pltpu.sync_copy(ix.at[pl.ds(b, ch)], idx_v)
            pltpu.async_copy(tbl.at[idx_v], rows_v, sem).wait()
            pltpu.sync_copy(rows_v, out.at[pl.ds(b, ch)])
            return carry

        lax.fori_loop(0, nch, chunk, 0)

    return k(table, idx)


def _sc_conv_agg(x, ea, src, rel, dst):
    @functools.partial(
        pl.kernel,
        out_type=jax.ShapeDtypeStruct((2 * P, D), jnp.float32),
        mesh=_mesh(),
        scratch_types=[
            pltpu.VMEM((CH,), jnp.int32),
            pltpu.VMEM((CH,), jnp.int32),
            pltpu.VMEM((CH,), jnp.int32),
            pltpu.VMEM((CH, D), jnp.float32),
            pltpu.VMEM((CH, D), jnp.float32),
            pltpu.VMEM_SHARED((P, D), jnp.float32),
            pltpu.SemaphoreType.DMA,
            pltpu.SemaphoreType.DMA,
            pltpu.SemaphoreType.DMA,
        ],
    )
    def k(xh, eah, srch, relh, dsth, outh,
          sidx, ridx, didx, xrb, erb, aggs, semx, seme, semi):
        cid = lax.axis_index("c")
        sid = lax.axis_index("s")
        wid = sid * NC + cid
        rows_per_tile = P // NS

        def zrow(r, carry):
            for j in range(D // 16):
                xrb[r, pl.ds(j * 16, 16)] = jnp.zeros((16,), jnp.float32)
            return carry

        lax.fori_loop(0, CH, zrow, 0)
        for t in range(rows_per_tile // CH):
            pltpu.sync_copy(xrb, aggs.at[pl.ds(sid * rows_per_tile + t * CH, CH)])

        eb = wid * EW
        plsc.subcore_barrier()

        def chunk(c, carry):
            off = eb + c * CH
            c1 = pltpu.async_copy(srch.at[pl.ds(off, CH)], sidx, semi)
            c2 = pltpu.async_copy(relh.at[pl.ds(off, CH)], ridx, semi)
            c3 = pltpu.async_copy(dsth.at[pl.ds(off, CH)], didx, semi)
            c1.wait()
            c2.wait()
            c3.wait()
            cx = pltpu.async_copy(xh.at[sidx], xrb, semx)
            ce = pltpu.async_copy(eah.at[ridx], erb, seme)
            cx.wait()
            ce.wait()

            def row(r, carry2):
                for j in range(D // 16):
                    s = pl.ds(j * 16, 16)
                    erb[r, s] = jnp.maximum(xrb[r, s] + erb[r, s], 0.0)
                return carry2

            lax.fori_loop(0, CH, row, 0)
            pltpu.sync_copy(erb, aggs.at[didx], add=True)
            return carry

        lax.fori_loop(0, NCH, chunk, 0)
        plsc.subcore_barrier()
        pltpu.sync_copy(aggs.at[pl.ds(sid * rows_per_tile, rows_per_tile)],
                        outh.at[pl.ds(cid * P + sid * rows_per_tile, rows_per_tile)])

    return k(x, ea, src, rel, dst)


def _sc_edge_head(xs12, x2, src, dst, cen):
    nc_ = cen.shape[0]
    cw = nc_ // NW

    @functools.partial(
        pl.kernel,
        out_type=[jax.ShapeDtypeStruct((E, HD), jnp.float32),
                  jax.ShapeDtypeStruct((nc_, D), jnp.float32)],
        mesh=_mesh(),
        scratch_types=[
            pltpu.VMEM((CH,), jnp.int32),
            pltpu.VMEM((CH,), jnp.int32),
            pltpu.VMEM((CH, D), jnp.float32),
            pltpu.VMEM((CH, D), jnp.float32),
            pltpu.VMEM((CH, HD), jnp.float32),
            pltpu.VMEM((cw,), jnp.int32),
            pltpu.VMEM((cw, D), jnp.float32),
            pltpu.SemaphoreType.DMA,
            pltpu.SemaphoreType.DMA,
            pltpu.SemaphoreType.DMA,
        ],
    )
    def k(x12h, xh, srch, dsth, ch, ech, xch,
          sidx, didx, ar, br, obuf, cbuf, crows, semx, seme, semi):
        wid = lax.axis_index("s") * NC + lax.axis_index("c")
        eb = wid * EW

        def chunk(c, carry):
            off = eb + c * CH
            c1 = pltpu.async_copy(srch.at[pl.ds(off, CH)], sidx, semi)
            c2 = pltpu.async_copy(dsth.at[pl.ds(off, CH)], didx, semi)
            c1.wait()
            c2.wait()
            ca = pltpu.async_copy(x12h.at[sidx], ar, semx)
            cb = pltpu.async_copy(x12h.at[didx], br, seme)
            ca.wait()
            cb.wait()

            def row(r, carry2):
                for j in range(HD // 16):
                    obuf[r, pl.ds(j * 16, 16)] = (
                        ar[r, pl.ds(j * 16, 16)] + br[r, pl.ds(HD + j * 16, 16)])
                return carry2

            lax.fori_loop(0, CH, row, 0)
            pltpu.sync_copy(obuf, ech.at[pl.ds(off, CH)])
            return carry

        lax.fori_loop(0, NCH, chunk, 0)

        cb_ = wid * cw
        pltpu.sync_copy(ch.at[pl.ds(cb_, cw)], cbuf)
        pltpu.async_copy(xh.at[cbuf], crows, semx).wait()
        pltpu.sync_copy(crows, xch.at[pl.ds(cb_, cw)])

    return k(xs12, x2, src, dst, cen)


def _tc_linear(x, w, b):
    m = x.shape[0]

    def body(xb, wb, bb, ob):
        ob[...] = jnp.dot(xb[...], wb[...],
                          preferred_element_type=jnp.float32) + bb[...]

    return pl.pallas_call(
        body,
        grid=(m // 1024,),
        in_specs=[pl.BlockSpec((1024, D), lambda i: (i, 0)),
                  pl.BlockSpec((D, D), lambda i: (0, 0)),
                  pl.BlockSpec((1, D), lambda i: (0, 0))],
        out_specs=pl.BlockSpec((1024, D), lambda i: (i, 0)),
        out_shape=jax.ShapeDtypeStruct((m, D), jnp.float32),
    )(x, w, b.reshape(1, D))


def _tc_conv_mm(x, p0, p1, w, b, do_relu):
    def body(xb, ab, cb, wb, bb, ob):
        t = jnp.dot(xb[...] + ab[...] + cb[...], wb[...],
                    preferred_element_type=jnp.float32) + bb[...]
        if do_relu:
            t = jnp.maximum(t, 0.0)
        ob[...] = t

    return pl.pallas_call(
        body,
        grid=(P // 1024,),
        in_specs=[pl.BlockSpec((1024, D), lambda i: (i, 0)),
                  pl.BlockSpec((1024, D), lambda i: (i, 0)),
                  pl.BlockSpec((1024, D), lambda i: (i, 0)),
                  pl.BlockSpec((D, D), lambda i: (0, 0)),
                  pl.BlockSpec((1, D), lambda i: (0, 0))],
        out_specs=pl.BlockSpec((1024, D), lambda i: (i, 0)),
        out_shape=jax.ShapeDtypeStruct((P, D), jnp.float32),
    )(x, p0, p1, w, b.reshape(1, D))


def _tc_conv2_heads(x, p0, p1, w2, b2, wh, bh, wn, bn):
    def body(xb, ab, cb, w2b, b2b, whb, bhb, wnb, bnb, x2o, ho, no):
        t = jnp.dot(xb[...] + ab[...] + cb[...], w2b[...],
                    preferred_element_type=jnp.float32) + b2b[...]
        x2o[...] = t
        ho[...] = jnp.dot(t, whb[...],
                          preferred_element_type=jnp.float32) + bhb[...]
        no[...] = jnp.dot(t, wnb[...],
                          preferred_element_type=jnp.float32) + bnb[...]

    return pl.pallas_call(
        body,
        grid=(P // 1024,),
        in_specs=[pl.BlockSpec((1024, D), lambda i: (i, 0)),
                  pl.BlockSpec((1024, D), lambda i: (i, 0)),
                  pl.BlockSpec((1024, D), lambda i: (i, 0)),
                  pl.BlockSpec((D, D), lambda i: (0, 0)),
                  pl.BlockSpec((1, D), lambda i: (0, 0)),
                  pl.BlockSpec((D, D), lambda i: (0, 0)),
                  pl.BlockSpec((1, D), lambda i: (0, 0)),
                  pl.BlockSpec((D, 16), lambda i: (0, 0)),
                  pl.BlockSpec((1, 16), lambda i: (0, 0))],
        out_specs=[pl.BlockSpec((1024, D), lambda i: (i, 0)),
                   pl.BlockSpec((1024, D), lambda i: (i, 0)),
                   pl.BlockSpec((1024, 16), lambda i: (i, 0))],
        out_shape=[jax.ShapeDtypeStruct((P, D), jnp.float32),
                   jax.ShapeDtypeStruct((P, D), jnp.float32),
                   jax.ShapeDtypeStruct((P, 16), jnp.float32)],
    )(x, p0, p1, w2, b2.reshape(1, D), wh, bh.reshape(1, D),
      wn, bn.reshape(1, 16))


def _tc_motif(xc, w, b):
    def body(xb, wb, bb, ob):
        ob[...] = jnp.dot(xb[...], wb[...],
                          preferred_element_type=jnp.float32) + bb[...]

    m = xc.shape[0]
    return pl.pallas_call(
        body,
        in_specs=[pl.BlockSpec((m, D), lambda: (0, 0)),
                  pl.BlockSpec((D, D), lambda: (0, 0)),
                  pl.BlockSpec((1, D), lambda: (0, 0))],
        out_specs=pl.BlockSpec((m, D), lambda: (0, 0)),
        out_shape=jax.ShapeDtypeStruct((m, D), jnp.float32),
    )(xc, w, b.reshape(1, D))


def kernel(node_ids, rel_ids, center_mol_idx, edge_index, node_table, rel_table,
           lin_W, lin_b, conv1_W, conv1_b, conv2_W, conv2_b,
           ec_W, ec_b, mo_W, mo_b, nc_W, nc_b):
    i32 = jnp.int32
    node_ids = node_ids.astype(i32)
    src = edge_index[0].astype(i32)
    dst = edge_index[1].astype(i32)
    rel = rel_ids.astype(i32)
    cen = center_mol_idx.astype(i32)

    idx_pad = jnp.concatenate([node_ids, jnp.zeros((P - N,), i32)])
    x0 = _sc_gather_rows(node_table, idx_pad)

    relp = jnp.zeros((1024, D), jnp.float32).at[:64].set(rel_table)
    proj = _tc_linear(jnp.concatenate([x0, relp], axis=0), lin_W, lin_b)
    x = proj[:P]
    ea = proj[P:P + 64]

    parts = _sc_conv_agg(x, ea, src, rel, dst)
    x1 = _tc_conv_mm(x, parts[:P], parts[P:], conv1_W, conv1_b, True)

    parts2 = _sc_conv_agg(x1, ea, src, rel, dst)
    wh = jnp.concatenate([ec_W[:D], ec_W[D:]], axis=1)
    bhd = jnp.zeros((D,), jnp.float32).at[:HD].set(ec_b)
    x2, xs12, ncls = _tc_conv2_heads(x1, parts2[:P], parts2[P:],
                                     conv2_W, conv2_b, wh, bhd, nc_W, nc_b)
    node_class = ncls[:N]

    edge_class, xc = _sc_edge_head(xs12, x2, src, dst, cen)

    mo_wp = jnp.zeros((D, D), jnp.float32).at[:, :85].set(mo_W)
    mo_bp = jnp.zeros((D,), jnp.float32).at[:85].set(mo_b)
    motif = _tc_motif(xc, mo_wp, mo_bp)[:, :85]

    return (edge_class, motif, node_class)


# speedup vs baseline: 6.6685x; 1.6733x over previous
import functools

import jax
import jax.numpy as jnp
from jax import lax
from jax.experimental import pallas as pl
from jax.experimental.pallas import tpu as pltpu
from jax.experimental.pallas import tpu_sc as plsc

N = 10000
E = 320000
P = 10240
D = 128
HD = 64
NC = 2
NS = 16
NW = NC * NS
EW = E // NW
CH = 80
NCH = EW // CH
IB = 5
NBLK = NCH // IB
SBCH = 2 * IB
NSB = NCH // SBCH


def _mesh():
    return plsc.VectorSubcoreMesh(core_axis_name="c", subcore_axis_name="s",
                                  num_cores=NC, num_subcores=NS)


def _sc_gather_rows(table, idx):
    n_out = idx.shape[0]
    d = table.shape[1]
    bw = n_out // NW
    ch = min(bw, CH)
    nch = bw // ch

    @functools.partial(
        pl.kernel,
        out_type=jax.ShapeDtypeStruct((n_out, d), jnp.float32),
        mesh=_mesh(),
        scratch_types=[pltpu.VMEM((ch,), jnp.int32),
                       pltpu.VMEM((ch, d), jnp.float32),
                       pltpu.SemaphoreType.DMA],
    )
    def k(tbl, ix, out, idx_v, rows_v, sem):
        base = (lax.axis_index("s") * NC + lax.axis_index("c")) * bw

        def chunk(c, carry):
            b = base + c * ch
            pltpu.sync_copy(ix.at[pl.ds(b, ch)], idx_v)
            pltpu.async_copy(tbl.at[idx_v], rows_v, sem).wait()
            pltpu.sync_copy(rows_v, out.at[pl.ds(b, ch)])
            return carry

        lax.fori_loop(0, nch, chunk, 0)

    return k(table, idx)


def _sc_conv_agg(x, ea, src2, rel2, dst2):
    @functools.partial(
        pl.kernel,
        out_type=jax.ShapeDtypeStruct((2 * P, D), jnp.float32),
        mesh=_mesh(),
        scratch_types=[
            pltpu.VMEM((SBCH, 1, CH), jnp.int32),
            pltpu.VMEM((SBCH, 1, CH), jnp.int32),
            pltpu.VMEM((SBCH, 1, CH), jnp.int32),
            pltpu.VMEM((CH, D), jnp.float32),
            pltpu.VMEM((CH, D), jnp.float32),
            pltpu.VMEM((CH, D), jnp.float32),
            pltpu.VMEM((CH, D), jnp.float32),
            pltpu.VMEM_SHARED((64, D), jnp.float32),
            pltpu.VMEM_SHARED((P, D), jnp.float32),
            pltpu.SemaphoreType.DMA,
            pltpu.SemaphoreType.DMA,
            pltpu.SemaphoreType.DMA,
            pltpu.SemaphoreType.DMA,
            pltpu.SemaphoreType.DMA,
            pltpu.SemaphoreType.DMA,
        ],
    )
    def k(xh, eah, srch, relh, dsth, outh,
          sblk, rblk, dblk, xrb0, xrb1, erb0, erb1, eas, aggs,
          semi0, semi1, semx0, semx1, seme0, seme1):
        cid = lax.axis_index("c")
        sid = lax.axis_index("s")
        wid = sid * NC + cid
        rows_per_tile = P // NS

        @pl.when(sid == 0)
        def _():
            pltpu.sync_copy(eah, eas)

        def zrow(r, carry):
            for j in range(D // 16):
                xrb0[r, pl.ds(j * 16, 16)] = jnp.zeros((16,), jnp.float32)
            return carry

        lax.fori_loop(0, CH, zrow, 0)
        for t in range(rows_per_tile // CH):
            pltpu.sync_copy(xrb0, aggs.at[pl.ds(sid * rows_per_tile + t * CH, CH)])
        plsc.subcore_barrier()

        crow0 = wid * NCH
        xrbs = (xrb0, xrb1)
        erbs = (erb0, erb1)
        semxs = (semx0, semx1)
        semes = (seme0, seme1)
        semis = (semi0, semi1)

        def refill(half, b):
            brow = crow0 + b * IB
            dsl = pl.ds(half * IB, IB)
            pltpu.async_copy(srch.at[pl.ds(brow, IB)], sblk.at[dsl], semis[half])
            pltpu.async_copy(relh.at[pl.ds(brow, IB)], rblk.at[dsl], semis[half])
            pltpu.async_copy(dsth.at[pl.ds(brow, IB)], dblk.at[dsl], semis[half])

        def refill_wait(half):
            dsl = pl.ds(half * IB, IB)
            base = pl.ds(crow0, IB)
            pltpu.make_async_copy(srch.at[base], sblk.at[dsl], semis[half]).wait()
            pltpu.make_async_copy(relh.at[base], rblk.at[dsl], semis[half]).wait()
            pltpu.make_async_copy(dsth.at[base], dblk.at[dsl], semis[half]).wait()

        def issue(jj, p):
            pltpu.async_copy(xh.at[sblk.at[jj, 0]], xrbs[p], semxs[p])
            pltpu.async_copy(eas.at[rblk.at[jj, 0]], erbs[p], semes[p])

        def wait(jj, p):
            pltpu.make_async_copy(xh.at[sblk.at[jj, 0]], xrbs[p], semxs[p]).wait()
            pltpu.make_async_copy(eas.at[rblk.at[jj, 0]], erbs[p], semes[p]).wait()

        def compute(p):
            xr, er = xrbs[p], erbs[p]

            def row2(r2, carry):
                r = 2 * r2
                for rr in (r, r + 1):
                    for jj in range(D // 16):
                        s = pl.ds(jj * 16, 16)
                        er[rr, s] = jnp.maximum(xr[rr, s] + er[rr, s], 0.0)
                return carry

            lax.fori_loop(0, CH // 2, row2, 0)

        def scatter(jj, p):
            pltpu.sync_copy(erbs[p], aggs.at[dblk.at[jj, 0]], add=True)

        refill(0, 0)
        refill(1, 1)
        refill_wait(0)
        issue(0, 0)

        def sblock(u, carry):
            for q in range(SBCH):
                p = q & 1
                if q == IB - 1:
                    refill_wait(1)
                if q == SBCH - 1:
                    refill_wait(0)
                issue((q + 1) % SBCH, 1 - p)
                wait(q, p)
                compute(p)
                scatter(q, p)
                if q == IB - 1:
                    refill(0, 2 * u + 2)
                if q == SBCH - 1:
                    b_next = 2 * u + 3

                    @pl.when(b_next < NBLK)
                    def _():
                        refill(1, b_next)
            return carry

        lax.fori_loop(0, NSB, sblock, 0)

        for q in range(IB):
            p = q & 1
            if q < IB - 1:
                issue(q + 1, 1 - p)
            wait(q, p)
            compute(p)
            scatter(q, p)

        plsc.subcore_barrier()
        pltpu.sync_copy(aggs.at[pl.ds(sid * rows_per_tile, rows_per_tile)],
                        outh.at[pl.ds(cid * P + sid * rows_per_tile, rows_per_tile)])

    return k(x, ea, src2, rel2, dst2)


def _sc_edge_head(xs12, x2, src2, dst2, cen):
    nc_ = cen.shape[0]
    cw = nc_ // NW

    @functools.partial(
        pl.kernel,
        out_type=[jax.ShapeDtypeStruct((E, HD), jnp.float32),
                  jax.ShapeDtypeStruct((nc_, D), jnp.float32)],
        mesh=_mesh(),
        scratch_types=[
            pltpu.VMEM((SBCH, 1, CH), jnp.int32),
            pltpu.VMEM((SBCH, 1, CH), jnp.int32),
            pltpu.VMEM((CH, D), jnp.float32),
            pltpu.VMEM((CH, D), jnp.float32),
            pltpu.VMEM((CH, D), jnp.float32),
            pltpu.VMEM((CH, D), jnp.float32),
            pltpu.VMEM((CH, HD), jnp.float32),
            pltpu.VMEM((CH, HD), jnp.float32),
            pltpu.VMEM((cw,), jnp.int32),
            pltpu.VMEM((cw, D), jnp.float32),
            pltpu.SemaphoreType.DMA,
            pltpu.SemaphoreType.DMA,
            pltpu.SemaphoreType.DMA,
            pltpu.SemaphoreType.DMA,
            pltpu.SemaphoreType.DMA,
            pltpu.SemaphoreType.DMA,
            pltpu.SemaphoreType.DMA,
            pltpu.SemaphoreType.DMA,
        ],
    )
    def k(x12h, xh, srch, dsth, ch, ech, xch,
          sblk, dblk, ar0, ar1, br0, br1, ob0, ob1, cbuf, crows,
          semi0, semi1, semx0, semx1, seme0, seme1, semo0, semo1):
        wid = lax.axis_index("s") * NC + lax.axis_index("c")
        crow0 = wid * NCH
        ars = (ar0, ar1)
        brs = (br0, br1)
        obs = (ob0, ob1)
        semxs = (semx0, semx1)
        semes = (seme0, seme1)
        semos = (semo0, semo1)
        semis = (semi0, semi1)

        def refill(half, b):
            brow = crow0 + b * IB
            dsl = pl.ds(half * IB, IB)
            pltpu.async_copy(srch.at[pl.ds(brow, IB)], sblk.at[dsl], semis[half])
            pltpu.async_copy(dsth.at[pl.ds(brow, IB)], dblk.at[dsl], semis[half])

        def refill_wait(half):
            dsl = pl.ds(half * IB, IB)
            base = pl.ds(crow0, IB)
            pltpu.make_async_copy(srch.at[base], sblk.at[dsl], semis[half]).wait()
            pltpu.make_async_copy(dsth.at[base], dblk.at[dsl], semis[half]).wait()

        def issue(jj, p):
            pltpu.async_copy(x12h.at[sblk.at[jj, 0]], ars[p], semxs[p])
            pltpu.async_copy(x12h.at[dblk.at[jj, 0]], brs[p], semes[p])

        def wait(jj, p):
            pltpu.make_async_copy(x12h.at[sblk.at[jj, 0]], ars[p], semxs[p]).wait()
            pltpu.make_async_copy(x12h.at[dblk.at[jj, 0]], brs[p], semes[p]).wait()

        def drain(p):
            pltpu.make_async_copy(
                obs[p], ech.at[pl.ds(crow0 * CH, CH)], semos[p]).wait()

        def compute_store(c, p):
            ar, br, ob = ars[p], brs[p], obs[p]

            def row2(r2, carry):
                r = 2 * r2
                for rr in (r, r + 1):
                    for jj in range(HD // 16):
                        ob[rr, pl.ds(jj * 16, 16)] = (
                            ar[rr, pl.ds(jj * 16, 16)]
                            + br[rr, pl.ds(HD + jj * 16, 16)])
                return carry

            lax.fori_loop(0, CH // 2, row2, 0)
            pltpu.async_copy(ob, ech.at[pl.ds(c * CH, CH)], semos[p])

        refill(0, 0)
        refill(1, 1)
        refill_wait(0)
        issue(0, 0)

        def sblock(u, carry):
            c0 = crow0 + SBCH * u
            for q in range(SBCH):
                p = q & 1
                if q == IB - 1:
                    refill_wait(1)
                if q == SBCH - 1:
                    refill_wait(0)
                issue((q + 1) % SBCH, 1 - p)
                wait(q, p)
                if q < 2:
                    @pl.when(u > 0)
                    def _():
                        drain(p)
                else:
                    drain(p)
                compute_store(c0 + q, p)
                if q == IB - 1:
                    refill(0, 2 * u + 2)
                if q == SBCH - 1:
                    b_next = 2 * u + 3

                    @pl.when(b_next < NBLK)
                    def _():
                        refill(1, b_next)
            return carry

        lax.fori_loop(0, NSB, sblock, 0)

        c0 = crow0 + NCH - IB
        for q in range(IB):
            p = q & 1
            if q < IB - 1:
                issue(q + 1, 1 - p)
            wait(q, p)
            drain(p)
            compute_store(c0 + q, p)
        drain(0)
        drain(1)

        cb_ = wid * cw
        pltpu.sync_copy(ch.at[pl.ds(cb_, cw)], cbuf)
        pltpu.async_copy(xh.at[cbuf], crows, semx0).wait()
        pltpu.sync_copy(crows, xch.at[pl.ds(cb_, cw)])

    return k(xs12, x2, src2, dst2, cen)


def _tc_linear(x, w, b):
    m = x.shape[0]

    def body(xb, wb, bb, ob):
        ob[...] = jnp.dot(xb[...], wb[...],
                          preferred_element_type=jnp.float32) + bb[...]

    return pl.pallas_call(
        body,
        grid=(m // 1024,),
        in_specs=[pl.BlockSpec((1024, D), lambda i: (i, 0)),
                  pl.BlockSpec((D, D), lambda i: (0, 0)),
                  pl.BlockSpec((1, D), lambda i: (0, 0))],
        out_specs=pl.BlockSpec((1024, D), lambda i: (i, 0)),
        out_shape=jax.ShapeDtypeStruct((m, D), jnp.float32),
    )(x, w, b.reshape(1, D))


def _tc_conv_mm(x, p0, p1, w, b, do_relu):
    def body(xb, ab, cb, wb, bb, ob):
        t = jnp.dot(xb[...] + ab[...] + cb[...], wb[...],
                    preferred_element_type=jnp.float32) + bb[...]
        if do_relu:
            t = jnp.maximum(t, 0.0)
        ob[...] = t

    return pl.pallas_call(
        body,
        grid=(P // 1024,),
        in_specs=[pl.BlockSpec((1024, D), lambda i: (i, 0)),
                  pl.BlockSpec((1024, D), lambda i: (i, 0)),
                  pl.BlockSpec((1024, D), lambda i: (i, 0)),
                  pl.BlockSpec((D, D), lambda i: (0, 0)),
                  pl.BlockSpec((1, D), lambda i: (0, 0))],
        out_specs=pl.BlockSpec((1024, D), lambda i: (i, 0)),
        out_shape=jax.ShapeDtypeStruct((P, D), jnp.float32),
    )(x, p0, p1, w, b.reshape(1, D))


def _tc_conv2_heads(x, p0, p1, w2, b2, wh, bh, wn, bn):
    def body(xb, ab, cb, w2b, b2b, whb, bhb, wnb, bnb, x2o, ho, no):
        t = jnp.dot(xb[...] + ab[...] + cb[...], w2b[...],
                    preferred_element_type=jnp.float32) + b2b[...]
        x2o[...] = t
        ho[...] = jnp.dot(t, whb[...],
                          preferred_element_type=jnp.float32) + bhb[...]
        no[...] = jnp.dot(t, wnb[...],
                          preferred_element_type=jnp.float32) + bnb[...]

    return pl.pallas_call(
        body,
        grid=(P // 1024,),
        in_specs=[pl.BlockSpec((1024, D), lambda i: (i, 0)),
                  pl.BlockSpec((1024, D), lambda i: (i, 0)),
                  pl.BlockSpec((1024, D), lambda i: (i, 0)),
                  pl.BlockSpec((D, D), lambda i: (0, 0)),
                  pl.BlockSpec((1, D), lambda i: (0, 0)),
                  pl.BlockSpec((D, D), lambda i: (0, 0)),
                  pl.BlockSpec((1, D), lambda i: (0, 0)),
                  pl.BlockSpec((D, 16), lambda i: (0, 0)),
                  pl.BlockSpec((1, 16), lambda i: (0, 0))],
        out_specs=[pl.BlockSpec((1024, D), lambda i: (i, 0)),
                   pl.BlockSpec((1024, D), lambda i: (i, 0)),
                   pl.BlockSpec((1024, 16), lambda i: (i, 0))],
        out_shape=[jax.ShapeDtypeStruct((P, D), jnp.float32),
                   jax.ShapeDtypeStruct((P, D), jnp.float32),
                   jax.ShapeDtypeStruct((P, 16), jnp.float32)],
    )(x, p0, p1, w2, b2.reshape(1, D), wh, bh.reshape(1, D),
      wn, bn.reshape(1, 16))


def _tc_motif(xc, w, b):
    def body(xb, wb, bb, ob):
        ob[...] = jnp.dot(xb[...], wb[...],
                          preferred_element_type=jnp.float32) + bb[...]

    m = xc.shape[0]
    return pl.pallas_call(
        body,
        in_specs=[pl.BlockSpec((m, D), lambda: (0, 0)),
                  pl.BlockSpec((D, D), lambda: (0, 0)),
                  pl.BlockSpec((1, D), lambda: (0, 0))],
        out_specs=pl.BlockSpec((m, D), lambda: (0, 0)),
        out_shape=jax.ShapeDtypeStruct((m, D), jnp.float32),
    )(xc, w, b.reshape(1, D))


def kernel(node_ids, rel_ids, center_mol_idx, edge_index, node_table, rel_table,
           lin_W, lin_b, conv1_W, conv1_b, conv2_W, conv2_b,
           ec_W, ec_b, mo_W, mo_b, nc_W, nc_b):
    i32 = jnp.int32
    node_ids = node_ids.astype(i32)
    src2 = edge_index[0].astype(i32).reshape(E // CH, 1, CH)
    dst2 = edge_index[1].astype(i32).reshape(E // CH, 1, CH)
    rel2 = rel_ids.astype(i32).reshape(E // CH, 1, CH)
    cen = center_mol_idx.astype(i32)

    idx_pad = jnp.concatenate([node_ids, jnp.zeros((P - N,), i32)])
    x0 = _sc_gather_rows(node_table, idx_pad)

    relp = jnp.zeros((1024, D), jnp.float32).at[:64].set(rel_table)
    proj = _tc_linear(jnp.concatenate([x0, relp], axis=0), lin_W, lin_b)
    x = proj[:P]
    ea = proj[P:P + 64]

    parts = _sc_conv_agg(x, ea, src2, rel2, dst2)
    x1 = _tc_conv_mm(x, parts[:P], parts[P:], conv1_W, conv1_b, True)

    parts2 = _sc_conv_agg(x1, ea, src2, rel2, dst2)
    wh = jnp.concatenate([ec_W[:D], ec_W[D:]], axis=1)
    bhd = jnp.zeros((D,), jnp.float32).at[:HD].set(ec_b)
    x2, xs12, ncls = _tc_conv2_heads(x1, parts2[:P], parts2[P:],
                                     conv2_W, conv2_b, wh, bhd, nc_W, nc_b)
    node_class = ncls[:N]

    edge_class, xc = _sc_edge_head(xs12, x2, src2, dst2, cen)

    mo_wp = jnp.zeros((D, D), jnp.float32).at[:, :85].set(mo_W)
    mo_bp = jnp.zeros((D,), jnp.float32).at[:85].set(mo_b)
    motif = _tc_motif(xc, mo_wp, mo_bp)[:, :85]

    return (edge_class, motif, node_class)
